# baseline (speedup 1.0000x reference)
import functools

import jax
import jax.numpy as jnp
from jax import lax
from jax.experimental import pallas as pl
from jax.experimental.pallas import tpu as pltpu
from jax.experimental.pallas import tpu_sc as plsc

N = 10000
E = 320000
G = 64
C = 10
HN = 8
HW = 16
H = 6
NS = 16
NPT = 640
NP = NS * NPT
NCHN = NPT // 128
ECH2 = 80
EPAD2 = 32 * ECH2 * 128
MAGIC = 1.0e9


def _rsqrt16(x):
    bits = lax.bitcast_convert_type(x, jnp.int32)
    y = lax.bitcast_convert_type(
        jnp.full((16,), 0x5F3759DF, jnp.int32) - (bits >> 1), jnp.float32)
    for _ in range(3):
        y = y * (1.5 - 0.5 * x * y * y)
    return y


_GD = lax.GatherDimensionNumbers(
    offset_dims=(), collapsed_slice_dims=(0,), start_index_map=(0,))


def _perm(v, idx16):
    return lax.gather(v, idx16.reshape(16, 1), _GD, (1,),
                      mode=lax.GatherScatterMode.PROMISE_IN_BOUNDS)


def _splat(v, k):
    return _perm(v, jnp.full((16,), k, jnp.int32))


def _sc_body(z1, srcq, dstq, batr, wst, bst,
             out, pout, pout7, flags,
             bufA, bufB, bufC, bufG7, bufT7, sums,
             srcv2, dstv2, batv, wv, bv, disv,
             tv8, gv8, pv8, pv8b, tv16, gv16, pv16, pv16b,
             cv16, zv8, onesv, mgv, fv, st0, st1, sw0, sw1,
             g0, g1, s0, s1, p0, p1):
    cid = lax.axis_index("c")
    sid = lax.axis_index("s")
    wid = cid * NS + sid
    oc = 1 - cid
    nbase = sid * NPT

    iota = lax.iota(jnp.int32, 16)
    col8 = iota & 7
    pair01 = iota >> 3
    spidx = [pair01 * 8 + k for k in range(H)]

    pltpu.sync_copy(srcq.at[wid], srcv2)
    pltpu.sync_copy(dstq.at[wid], dstv2)
    pltpu.sync_copy(batr.at[sid], batv)
    pltpu.sync_copy(wst, wv)
    pltpu.sync_copy(bst, bv)

    @pl.loop(0, 128)
    def _fill(i):
        cv16[i] = jnp.zeros((HW,), jnp.float32)

    @pl.loop(0, 64, unroll=2)
    def _fill8(p):
        ridx = 2 * p + pair01
        plsc.store_scatter(zv8, [ridx, col8], jnp.zeros((16,), jnp.float32))
        plsc.store_scatter(onesv, [ridx, col8], jnp.ones((16,), jnp.float32))

    mgv[0] = jnp.full((16,), MAGIC, jnp.float32)

    for c in range(NCHN):
        pltpu.sync_copy(zv8, bufC.at[pl.ds(nbase + c * 128, 128)])

    @pl.when(sid == 0)
    def _zero_sums():
        pltpu.sync_copy(cv16.at[pl.ds(0, 72)], sums)

    plsc.subcore_barrier()

    def poll(row):
        @pl.when(sid == 0)
        def _p():
            def cond(v):
                return v != MAGIC

            def body(v):
                del v
                pltpu.sync_copy(flags.at[oc].at[pl.ds(row, 1)], fv)
                return jnp.min(fv[0])

            lax.while_loop(cond, body, jnp.float32(0.0))

        plsc.subcore_barrier()

    def zero_flag(row):
        @pl.when(sid == 0)
        def _z():
            pltpu.sync_copy(cv16.at[pl.ds(0, 1)], flags.at[oc].at[pl.ds(row, 1)])

    def publish(bufAcc, dst_slab, row):
        pltpu.sync_copy(bufAcc.at[pl.ds(nbase, NPT)],
                        dst_slab.at[pl.ds(nbase, NPT)])
        plsc.subcore_barrier()

        @pl.when(sid == 0)
        def _flag():
            pltpu.sync_copy(mgv, flags.at[cid].at[pl.ds(row, 1)])

        poll(row)

    @pl.loop(0, ECH2, step=2)
    def _deg(j):
        c0 = pltpu.async_copy(onesv, bufC.at[dstv2.at[j]], s0, add=True)
        c1 = pltpu.async_copy(onesv, bufC.at[dstv2.at[j + 1]], s1, add=True)
        c0.wait()
        c1.wait()

    plsc.subcore_barrier()
    publish(bufC, pout.at[cid].at[6], 6)

    pslab6 = pout.at[oc].at[6]
    for c in range(NCHN):
        pltpu.sync_copy(bufC.at[pl.ds(nbase + c * 128, 128)], tv8)
        pltpu.sync_copy(pslab6.at[pl.ds(nbase + c * 128, 128)], pv8)

        @pl.loop(0, 64, unroll=2)
        def _dis(p):
            ridx = 2 * p + pair01
            dp = (plsc.load_gather(tv8, [ridx, col8])
                  + plsc.load_gather(pv8, [ridx, col8]))
            disv[c * 64 + p] = _rsqrt16(dp + 1.0)

    def pointwise(l, bufT, bufG, bufAcc):
        if l >= 2:
            wrows = [wv[(l - 2) * H + k] for k in range(H)]
            brow = bv[l - 2]
            pslab = pout.at[oc].at[l - 2]
            pltpu.async_copy(pslab.at[pl.ds(nbase, 128)], pv8, p0)
        for c in range(NCHN):
            rng = pl.ds(nbase + c * 128, 128)
            pv, ps = (pv8, p0) if c % 2 == 0 else (pv8b, p1)
            if l == 1:
                pltpu.sync_copy(z1.at[rng], tv8)
            else:
                if c + 1 < NCHN:
                    nv, nps = (pv8b, p1) if c % 2 == 0 else (pv8, p0)
                    pltpu.async_copy(
                        pslab.at[pl.ds(nbase + (c + 1) * 128, 128)], nv, nps)
                pltpu.make_async_copy(pslab.at[rng], pv, ps).wait()
                pltpu.sync_copy(bufT.at[rng], tv8)

            @pl.loop(0, 64, unroll=2)
            def _row(p):
                d = disv[c * 64 + p]
                ridx = 2 * p + pair01
                t = plsc.load_gather(tv8, [ridx, col8])
                if l == 1:
                    g = d * t
                else:
                    t = t + plsc.load_gather(pv, [ridx, col8])
                    h = jnp.maximum(d * t + brow, 0.0)
                    z = _perm(h, spidx[0]) * wrows[0]
                    for k in range(1, H):
                        z = z + _perm(h, spidx[k]) * wrows[k]
                    g = d * z
                plsc.store_scatter(gv8, [ridx, col8], g)

            pltpu.sync_copy(gv8, bufG.at[rng])

            @pl.when(cid == 0)
            def _acc_self():
                pltpu.sync_copy(gv8, bufAcc.at[rng])

            @pl.when(cid == 1)
            def _acc_zero():
                pltpu.sync_copy(zv8, bufAcc.at[rng])

    def pointwise7():
        wrows = [wv[30 + k] for k in range(H)]
        brow = bv[5]
        pslab = pout.at[oc].at[5]
        pltpu.async_copy(pslab.at[pl.ds(nbase, 128)], pv8, p0)
        for c in range(NCHN):
            rng = pl.ds(nbase + c * 128, 128)
            pv, ps = (pv8, p0) if c % 2 == 0 else (pv8b, p1)
            if c + 1 < NCHN:
                nv, nps = (pv8b, p1) if c % 2 == 0 else (pv8, p0)
                pltpu.async_copy(
                    pslab.at[pl.ds(nbase + (c + 1) * 128, 128)], nv, nps)
            pltpu.make_async_copy(pslab.at[rng], pv, ps).wait()
            pltpu.sync_copy(bufC.at[rng], tv8)

            @pl.loop(0, 64, unroll=2)
            def _row(p):
                d = disv[c * 64 + p]
                ridx = 2 * p + pair01
                t = (plsc.load_gather(tv8, [ridx, col8])
                     + plsc.load_gather(pv, [ridx, col8]))
                h = jnp.maximum(d * t + brow, 0.0)
                z0 = _splat(h, 0) * wrows[0]
                z1_ = _splat(h, 8) * wrows[0]
                for k in range(1, H):
                    z0 = z0 + _splat(h, k) * wrows[k]
                    z1_ = z1_ + _splat(h, 8 + k) * wrows[k]
                gv16[2 * p] = _splat(d, 0) * z0
                gv16[2 * p + 1] = _splat(d, 8) * z1_

            pltpu.sync_copy(gv16, bufG7.at[rng])

            @pl.when(cid == 0)
            def _acc_self():
                pltpu.sync_copy(gv16, bufT7.at[rng])

            @pl.when(cid == 1)
            def _acc_zero():
                pltpu.sync_copy(cv16, bufT7.at[rng])

    def edge(bufG, bufAcc, sa, sb):
        pltpu.async_copy(bufG.at[srcv2.at[0]], sa, g0)

        @pl.loop(0, ECH2, step=2)
        def _e(j):
            pltpu.make_async_copy(bufG.at[srcv2.at[j]], sa, g0).wait()
            gn = pltpu.async_copy(bufG.at[srcv2.at[j + 1]], sb, g1)
            sc0 = pltpu.async_copy(sa, bufAcc.at[dstv2.at[j]], s0, add=True)
            gn.wait()
            sc1 = pltpu.async_copy(sb, bufAcc.at[dstv2.at[j + 1]], s1,
                                   add=True)
            sc0.wait()

            @pl.when(j + 2 < ECH2)
            def _next():
                pltpu.async_copy(bufG.at[srcv2.at[j + 2]], sa, g0)

            sc1.wait()

    seq = [
        (1, None, bufA, bufB),
        (2, bufB, bufC, bufA),
        (3, bufA, bufB, bufC),
        (4, bufC, bufA, bufB),
        (5, bufB, bufC, bufA),
        (6, bufA, bufB, bufC),
    ]
    for l, bufT, bufG, bufAcc in seq:
        pointwise(l, bufT, bufG, bufAcc)
        plsc.subcore_barrier()
        zero_flag(l - 2 if l >= 2 else 6)
        edge(bufG, bufAcc, st0, st1)
        plsc.subcore_barrier()
        publish(bufAcc, pout.at[cid].at[l - 1], l - 1)

    pointwise7()
    plsc.subcore_barrier()
    zero_flag(5)
    edge(bufG7, bufT7, sw0, sw1)
    plsc.subcore_barrier()
    publish(bufT7, pout7.at[cid], 7)

    brow = bv[6]
    pslab7 = pout7.at[oc]
    pltpu.async_copy(pslab7.at[pl.ds(nbase, 128)], pv16, p0)
    for c in range(NCHN):
        rng = pl.ds(nbase + c * 128, 128)
        pv, ps = (pv16, p0) if c % 2 == 0 else (pv16b, p1)
        if c + 1 < NCHN:
            nv, nps = (pv16b, p1) if c % 2 == 0 else (pv16, p0)
            pltpu.async_copy(
                pslab7.at[pl.ds(nbase + (c + 1) * 128, 128)], nv, nps)
        pltpu.make_async_copy(pslab7.at[rng], pv, ps).wait()
        pltpu.sync_copy(bufT7.at[rng], tv16)

        @pl.loop(0, 64, unroll=2)
        def _row(p):
            d = disv[c * 64 + p]
            t0 = tv16[2 * p] + pv[2 * p]
            t1 = tv16[2 * p + 1] + pv[2 * p + 1]
            gv16[2 * p] = jnp.maximum(_splat(d, 0) * t0 + brow, 0.0)
            gv16[2 * p + 1] = jnp.maximum(_splat(d, 8) * t1 + brow, 0.0)

        pltpu.sync_copy(gv16, sums.at[batv.at[c]], add=True)

    plsc.subcore_barrier()
    zero_flag(7)

    @pl.when(jnp.logical_and(cid == 0, sid == 0))
    def _write():
        pltpu.sync_copy(sums.at[pl.ds(0, G)], out)


_MESH = plsc.VectorSubcoreMesh(core_axis_name="c", subcore_axis_name="s",
                               num_cores=2, num_subcores=NS)

_sc_kernel = functools.partial(
    pl.kernel,
    out_type=(
        jax.ShapeDtypeStruct((G, HW), jnp.float32),
        jax.ShapeDtypeStruct((2, 7, NP, HN), jnp.float32),
        jax.ShapeDtypeStruct((2, NP, HW), jnp.float32),
        jax.ShapeDtypeStruct((2, 8, HW), jnp.float32),
    ),
    mesh=_MESH,
    compiler_params=pltpu.CompilerParams(use_tc_tiling_on_sc=False,
                                         needs_layout_passes=False),
    scratch_types=[
        pltpu.VMEM_SHARED((NP, HN), jnp.float32),
        pltpu.VMEM_SHARED((NP, HN), jnp.float32),
        pltpu.VMEM_SHARED((NP, HN), jnp.float32),
        pltpu.VMEM_SHARED((NP, HW), jnp.float32),
        pltpu.VMEM_SHARED((NP, HW), jnp.float32),
        pltpu.VMEM_SHARED((72, HW), jnp.float32),
        pltpu.VMEM((ECH2, 128), jnp.int32),
        pltpu.VMEM((ECH2, 128), jnp.int32),
        pltpu.VMEM((NCHN, 128), jnp.int32),
        pltpu.VMEM((36, HW), jnp.float32),
        pltpu.VMEM((7, HW), jnp.float32),
        pltpu.VMEM((NPT // 2, HW), jnp.float32),
        pltpu.VMEM((128, HN), jnp.float32),
        pltpu.VMEM((128, HN), jnp.float32),
        pltpu.VMEM((128, HN), jnp.float32),
        pltpu.VMEM((128, HN), jnp.float32),
        pltpu.VMEM((128, HW), jnp.float32),
        pltpu.VMEM((128, HW), jnp.float32),
        pltpu.VMEM((128, HW), jnp.float32),
        pltpu.VMEM((128, HW), jnp.float32),
        pltpu.VMEM((128, HW), jnp.float32),
        pltpu.VMEM((128, HN), jnp.float32),
        pltpu.VMEM((128, HN), jnp.float32),
        pltpu.VMEM((1, HW), jnp.float32),
        pltpu.VMEM((1, HW), jnp.float32),
        pltpu.VMEM((128, HN), jnp.float32),
        pltpu.VMEM((128, HN), jnp.float32),
        pltpu.VMEM((128, HW), jnp.float32),
        pltpu.VMEM((128, HW), jnp.float32),
        pltpu.SemaphoreType.DMA,
        pltpu.SemaphoreType.DMA,
        pltpu.SemaphoreType.DMA,
        pltpu.SemaphoreType.DMA,
        pltpu.SemaphoreType.DMA,
        pltpu.SemaphoreType.DMA,
    ],
)(_sc_body)


def _mm_body(x_ref, w_ref, o_ref):
    o_ref[...] = jnp.dot(x_ref[...], w_ref[...],
                         preferred_element_type=jnp.float32)


def _mm_tc(xp, w):
    return pl.pallas_call(
        _mm_body,
        out_shape=jax.ShapeDtypeStruct((NP, HN), jnp.float32),
    )(xp, w)


def _tail_body(sums_ref, batch_ref, out_ref):
    sums = sums_ref[...]
    batch = batch_ref[...]
    gid = jax.lax.broadcasted_iota(jnp.int32, (G, N), 0)
    cnt = jnp.sum((batch[None, :] == gid).astype(jnp.float32), axis=1)
    mean = sums[:, :C] / jnp.maximum(cnt, 1.0)[:, None]
    m = jnp.max(mean, axis=1, keepdims=True)
    e = jnp.exp(mean - m)
    lse = jnp.log(jnp.sum(e, axis=1, keepdims=True))
    out_ref[...] = mean - m - lse


def _tail(sums, batch):
    return pl.pallas_call(
        _tail_body,
        out_shape=jax.ShapeDtypeStruct((G, C), jnp.float32),
    )(sums, batch)


def kernel(x, edge_index, batch, W1, b1, W2, b2, W3, b3, W4, b4, W5, b5,
           W6, b6, Wf, bf):
    src = edge_index[0].astype(jnp.int32)
    dst = edge_index[1].astype(jnp.int32)
    bat = batch.astype(jnp.int32)

    xp = jnp.pad(x, ((0, NP - N), (0, 0)))
    W1p = jnp.pad(W1, ((0, 0), (0, HN - H)))
    z1 = _mm_tc(xp, W1p)

    srcq = jnp.pad(src, (0, EPAD2 - E), constant_values=N).reshape(32, ECH2, 128)
    dstq = jnp.pad(dst, (0, EPAD2 - E), constant_values=N).reshape(32, ECH2, 128)
    batp = jnp.pad(bat, (0, NP - N), constant_values=G).reshape(NS, NCHN, 128)

    def dup8(W):
        Wp = jnp.pad(W, ((0, 0), (0, HN - W.shape[1])))
        return jnp.concatenate([Wp, Wp], axis=1)

    Wst = jnp.concatenate(
        [dup8(W) for W in (W2, W3, W4, W5, W6)]
        + [jnp.pad(Wf, ((0, 0), (0, HW - C)))], axis=0)

    def bdup(b):
        bp = jnp.pad(b, (0, HN - b.shape[0]))
        return jnp.concatenate([bp, bp])

    bst = jnp.stack([bdup(b) for b in (b1, b2, b3, b4, b5, b6)]
                    + [jnp.pad(bf, (0, HW - C))])

    sums, _, _, _ = _sc_kernel(z1, srcq, dstq, batp, Wst, bst)
    return _tail(sums, bat)


# speedup vs baseline: 1.0131x; 1.0131x over previous
import functools

import jax
import jax.numpy as jnp
from jax import lax
from jax.experimental import pallas as pl
from jax.experimental.pallas import tpu as pltpu
from jax.experimental.pallas import tpu_sc as plsc

N = 10000
E = 320000
G = 64
C = 10
HN = 8
HW = 16
H = 6
NS = 16
NPT = 640
NP = NS * NPT
NCHN = NPT // 128
ECH2 = 80
EPAD2 = 32 * ECH2 * 128
MAGIC = 1.0e9


def _rsqrt16(x):
    bits = lax.bitcast_convert_type(x, jnp.int32)
    y = lax.bitcast_convert_type(
        jnp.full((16,), 0x5F3759DF, jnp.int32) - (bits >> 1), jnp.float32)
    for _ in range(3):
        y = y * (1.5 - 0.5 * x * y * y)
    return y


_GD = lax.GatherDimensionNumbers(
    offset_dims=(), collapsed_slice_dims=(0,), start_index_map=(0,))


def _perm(v, idx16):
    return lax.gather(v, idx16.reshape(16, 1), _GD, (1,),
                      mode=lax.GatherScatterMode.PROMISE_IN_BOUNDS)


def _splat(v, k):
    return _perm(v, jnp.full((16,), k, jnp.int32))


def _sc_body(z1, srcq, dstq, batr, wst, bst,
             out, pout, pout7, flags,
             bufA, bufB, bufC, bufG7, bufT7, sums,
             srcv2, dstv2, batv, wv, bv, disv,
             tv8, gv8, pv8, pv8b, tv16, gv16, pv16, pv16b,
             cv16, zv8, onesv, mgv, fv, st0, st1, sw0, sw1,
             g0, g1, s0, s1, p0, p1):
    cid = lax.axis_index("c")
    sid = lax.axis_index("s")
    wid = cid * NS + sid
    oc = 1 - cid
    nbase = sid * NPT

    iota = lax.iota(jnp.int32, 16)
    col8 = iota & 7
    pair01 = iota >> 3
    spidx = [pair01 * 8 + k for k in range(H)]

    pltpu.sync_copy(srcq.at[wid], srcv2)
    pltpu.sync_copy(dstq.at[wid], dstv2)
    pltpu.sync_copy(batr.at[sid], batv)
    pltpu.sync_copy(wst, wv)
    pltpu.sync_copy(bst, bv)

    @pl.loop(0, 128)
    def _fill(i):
        cv16[i] = jnp.zeros((HW,), jnp.float32)

    @pl.loop(0, 64)
    def _fill8(p):
        ridx = 2 * p + pair01
        plsc.store_scatter(zv8, [ridx, col8], jnp.zeros((16,), jnp.float32))
        plsc.store_scatter(onesv, [ridx, col8], jnp.ones((16,), jnp.float32))

    mgv[0] = jnp.full((16,), MAGIC, jnp.float32)

    for c in range(NCHN):
        pltpu.sync_copy(zv8, bufC.at[pl.ds(nbase + c * 128, 128)])

    @pl.when(sid == 0)
    def _zero_sums():
        pltpu.sync_copy(cv16.at[pl.ds(0, 72)], sums)

    plsc.subcore_barrier()

    def poll(row):
        @pl.when(sid == 0)
        def _p():
            def cond(v):
                return v != MAGIC

            def body(v):
                del v
                pltpu.sync_copy(flags.at[oc].at[pl.ds(row, 1)], fv)
                return jnp.min(fv[0])

            lax.while_loop(cond, body, jnp.float32(0.0))

        plsc.subcore_barrier()

    def zero_flag(row):
        @pl.when(sid == 0)
        def _z():
            pltpu.sync_copy(cv16.at[pl.ds(0, 1)], flags.at[oc].at[pl.ds(row, 1)])

    def publish(bufAcc, dst_slab, row):
        pltpu.sync_copy(bufAcc.at[pl.ds(nbase, NPT)],
                        dst_slab.at[pl.ds(nbase, NPT)])
        plsc.subcore_barrier()

        @pl.when(sid == 0)
        def _flag():
            pltpu.sync_copy(mgv, flags.at[cid].at[pl.ds(row, 1)])

        poll(row)

    @pl.loop(0, ECH2, step=2)
    def _deg(j):
        c0 = pltpu.async_copy(onesv, bufC.at[dstv2.at[j]], s0, add=True)
        c1 = pltpu.async_copy(onesv, bufC.at[dstv2.at[j + 1]], s1, add=True)
        c0.wait()
        c1.wait()

    plsc.subcore_barrier()
    publish(bufC, pout.at[cid].at[6], 6)

    pslab6 = pout.at[oc].at[6]
    for c in range(NCHN):
        pltpu.sync_copy(bufC.at[pl.ds(nbase + c * 128, 128)], tv8)
        pltpu.sync_copy(pslab6.at[pl.ds(nbase + c * 128, 128)], pv8)

        @pl.loop(0, 64)
        def _dis(p):
            ridx = 2 * p + pair01
            dp = (plsc.load_gather(tv8, [ridx, col8])
                  + plsc.load_gather(pv8, [ridx, col8]))
            disv[c * 64 + p] = _rsqrt16(dp + 1.0)

    def pointwise(l, bufT, bufG, bufAcc):
        if l >= 2:
            wrows = [wv[(l - 2) * H + k] for k in range(H)]
            brow = bv[l - 2]
            pslab = pout.at[oc].at[l - 2]
            pltpu.async_copy(pslab.at[pl.ds(nbase, 128)], pv8, p0)
        for c in range(NCHN):
            rng = pl.ds(nbase + c * 128, 128)
            pv, ps = (pv8, p0) if c % 2 == 0 else (pv8b, p1)
            if l == 1:
                pltpu.sync_copy(z1.at[rng], tv8)
            else:
                if c + 1 < NCHN:
                    nv, nps = (pv8b, p1) if c % 2 == 0 else (pv8, p0)
                    pltpu.async_copy(
                        pslab.at[pl.ds(nbase + (c + 1) * 128, 128)], nv, nps)
                pltpu.make_async_copy(pslab.at[rng], pv, ps).wait()
                pltpu.sync_copy(bufT.at[rng], tv8)

            @pl.loop(0, 64)
            def _row(p):
                d = disv[c * 64 + p]
                ridx = 2 * p + pair01
                t = plsc.load_gather(tv8, [ridx, col8])
                if l == 1:
                    g = d * t
                else:
                    t = t + plsc.load_gather(pv, [ridx, col8])
                    h = jnp.maximum(d * t + brow, 0.0)
                    z = _perm(h, spidx[0]) * wrows[0]
                    for k in range(1, H):
                        z = z + _perm(h, spidx[k]) * wrows[k]
                    g = d * z
                plsc.store_scatter(gv8, [ridx, col8], g)

            pltpu.sync_copy(gv8, bufG.at[rng])

            @pl.when(cid == 0)
            def _acc_self():
                pltpu.sync_copy(gv8, bufAcc.at[rng])

            @pl.when(cid == 1)
            def _acc_zero():
                pltpu.sync_copy(zv8, bufAcc.at[rng])

    def pointwise7():
        wrows = [wv[30 + k] for k in range(H)]
        brow = bv[5]
        pslab = pout.at[oc].at[5]
        pltpu.async_copy(pslab.at[pl.ds(nbase, 128)], pv8, p0)
        for c in range(NCHN):
            rng = pl.ds(nbase + c * 128, 128)
            pv, ps = (pv8, p0) if c % 2 == 0 else (pv8b, p1)
            if c + 1 < NCHN:
                nv, nps = (pv8b, p1) if c % 2 == 0 else (pv8, p0)
                pltpu.async_copy(
                    pslab.at[pl.ds(nbase + (c + 1) * 128, 128)], nv, nps)
            pltpu.make_async_copy(pslab.at[rng], pv, ps).wait()
            pltpu.sync_copy(bufC.at[rng], tv8)

            @pl.loop(0, 64)
            def _row(p):
                d = disv[c * 64 + p]
                ridx = 2 * p + pair01
                t = (plsc.load_gather(tv8, [ridx, col8])
                     + plsc.load_gather(pv, [ridx, col8]))
                h = jnp.maximum(d * t + brow, 0.0)
                z0 = _splat(h, 0) * wrows[0]
                z1_ = _splat(h, 8) * wrows[0]
                for k in range(1, H):
                    z0 = z0 + _splat(h, k) * wrows[k]
                    z1_ = z1_ + _splat(h, 8 + k) * wrows[k]
                gv16[2 * p] = _splat(d, 0) * z0
                gv16[2 * p + 1] = _splat(d, 8) * z1_

            pltpu.sync_copy(gv16, bufG7.at[rng])

            @pl.when(cid == 0)
            def _acc_self():
                pltpu.sync_copy(gv16, bufT7.at[rng])

            @pl.when(cid == 1)
            def _acc_zero():
                pltpu.sync_copy(cv16, bufT7.at[rng])

    def edge(bufG, bufAcc, sa, sb):
        pltpu.async_copy(bufG.at[srcv2.at[0]], sa, g0)

        @pl.loop(0, ECH2, step=2)
        def _e(j):
            pltpu.make_async_copy(bufG.at[srcv2.at[j]], sa, g0).wait()
            gn = pltpu.async_copy(bufG.at[srcv2.at[j + 1]], sb, g1)
            sc0 = pltpu.async_copy(sa, bufAcc.at[dstv2.at[j]], s0, add=True)
            gn.wait()
            sc1 = pltpu.async_copy(sb, bufAcc.at[dstv2.at[j + 1]], s1,
                                   add=True)
            sc0.wait()

            @pl.when(j + 2 < ECH2)
            def _next():
                pltpu.async_copy(bufG.at[srcv2.at[j + 2]], sa, g0)

            sc1.wait()

    seq = [
        (1, None, bufA, bufB),
        (2, bufB, bufC, bufA),
        (3, bufA, bufB, bufC),
        (4, bufC, bufA, bufB),
        (5, bufB, bufC, bufA),
        (6, bufA, bufB, bufC),
    ]
    for l, bufT, bufG, bufAcc in seq:
        pointwise(l, bufT, bufG, bufAcc)
        plsc.subcore_barrier()
        zero_flag(l - 2 if l >= 2 else 6)
        edge(bufG, bufAcc, st0, st1)
        plsc.subcore_barrier()
        publish(bufAcc, pout.at[cid].at[l - 1], l - 1)

    pointwise7()
    plsc.subcore_barrier()
    zero_flag(5)
    edge(bufG7, bufT7, sw0, sw1)
    plsc.subcore_barrier()
    publish(bufT7, pout7.at[cid], 7)

    brow = bv[6]
    pslab7 = pout7.at[oc]
    pltpu.async_copy(pslab7.at[pl.ds(nbase, 128)], pv16, p0)
    for c in range(NCHN):
        rng = pl.ds(nbase + c * 128, 128)
        pv, ps = (pv16, p0) if c % 2 == 0 else (pv16b, p1)
        if c + 1 < NCHN:
            nv, nps = (pv16b, p1) if c % 2 == 0 else (pv16, p0)
            pltpu.async_copy(
                pslab7.at[pl.ds(nbase + (c + 1) * 128, 128)], nv, nps)
        pltpu.make_async_copy(pslab7.at[rng], pv, ps).wait()
        pltpu.sync_copy(bufT7.at[rng], tv16)

        @pl.loop(0, 64)
        def _row(p):
            d = disv[c * 64 + p]
            t0 = tv16[2 * p] + pv[2 * p]
            t1 = tv16[2 * p + 1] + pv[2 * p + 1]
            gv16[2 * p] = jnp.maximum(_splat(d, 0) * t0 + brow, 0.0)
            gv16[2 * p + 1] = jnp.maximum(_splat(d, 8) * t1 + brow, 0.0)

        pltpu.sync_copy(gv16, sums.at[batv.at[c]], add=True)

    plsc.subcore_barrier()
    zero_flag(7)

    @pl.when(jnp.logical_and(cid == 0, sid == 0))
    def _write():
        pltpu.sync_copy(sums.at[pl.ds(0, G)], out)


_MESH = plsc.VectorSubcoreMesh(core_axis_name="c", subcore_axis_name="s",
                               num_cores=2, num_subcores=NS)

_sc_kernel = functools.partial(
    pl.kernel,
    out_type=(
        jax.ShapeDtypeStruct((G, HW), jnp.float32),
        jax.ShapeDtypeStruct((2, 7, NP, HN), jnp.float32),
        jax.ShapeDtypeStruct((2, NP, HW), jnp.float32),
        jax.ShapeDtypeStruct((2, 8, HW), jnp.float32),
    ),
    mesh=_MESH,
    compiler_params=pltpu.CompilerParams(use_tc_tiling_on_sc=False,
                                         needs_layout_passes=False),
    scratch_types=[
        pltpu.VMEM_SHARED((NP, HN), jnp.float32),
        pltpu.VMEM_SHARED((NP, HN), jnp.float32),
        pltpu.VMEM_SHARED((NP, HN), jnp.float32),
        pltpu.VMEM_SHARED((NP, HW), jnp.float32),
        pltpu.VMEM_SHARED((NP, HW), jnp.float32),
        pltpu.VMEM_SHARED((72, HW), jnp.float32),
        pltpu.VMEM((ECH2, 128), jnp.int32),
        pltpu.VMEM((ECH2, 128), jnp.int32),
        pltpu.VMEM((NCHN, 128), jnp.int32),
        pltpu.VMEM((36, HW), jnp.float32),
        pltpu.VMEM((7, HW), jnp.float32),
        pltpu.VMEM((NPT // 2, HW), jnp.float32),
        pltpu.VMEM((128, HN), jnp.float32),
        pltpu.VMEM((128, HN), jnp.float32),
        pltpu.VMEM((128, HN), jnp.float32),
        pltpu.VMEM((128, HN), jnp.float32),
        pltpu.VMEM((128, HW), jnp.float32),
        pltpu.VMEM((128, HW), jnp.float32),
        pltpu.VMEM((128, HW), jnp.float32),
        pltpu.VMEM((128, HW), jnp.float32),
        pltpu.VMEM((128, HW), jnp.float32),
        pltpu.VMEM((128, HN), jnp.float32),
        pltpu.VMEM((128, HN), jnp.float32),
        pltpu.VMEM((1, HW), jnp.float32),
        pltpu.VMEM((1, HW), jnp.float32),
        pltpu.VMEM((128, HN), jnp.float32),
        pltpu.VMEM((128, HN), jnp.float32),
        pltpu.VMEM((128, HW), jnp.float32),
        pltpu.VMEM((128, HW), jnp.float32),
        pltpu.SemaphoreType.DMA,
        pltpu.SemaphoreType.DMA,
        pltpu.SemaphoreType.DMA,
        pltpu.SemaphoreType.DMA,
        pltpu.SemaphoreType.DMA,
        pltpu.SemaphoreType.DMA,
    ],
)(_sc_body)


def _mm_body(x_ref, w_ref, o_ref):
    o_ref[...] = jnp.dot(x_ref[...], w_ref[...],
                         preferred_element_type=jnp.float32)


def _mm_tc(xp, w):
    return pl.pallas_call(
        _mm_body,
        out_shape=jax.ShapeDtypeStruct((NP, HN), jnp.float32),
    )(xp, w)


def _tail_body(sums_ref, batch_ref, out_ref):
    sums = sums_ref[...]
    batch = batch_ref[...]
    gid = jax.lax.broadcasted_iota(jnp.int32, (G, N), 0)
    cnt = jnp.sum((batch[None, :] == gid).astype(jnp.float32), axis=1)
    mean = sums[:, :C] / jnp.maximum(cnt, 1.0)[:, None]
    m = jnp.max(mean, axis=1, keepdims=True)
    e = jnp.exp(mean - m)
    lse = jnp.log(jnp.sum(e, axis=1, keepdims=True))
    out_ref[...] = mean - m - lse


def _tail(sums, batch):
    return pl.pallas_call(
        _tail_body,
        out_shape=jax.ShapeDtypeStruct((G, C), jnp.float32),
    )(sums, batch)


def kernel(x, edge_index, batch, W1, b1, W2, b2, W3, b3, W4, b4, W5, b5,
           W6, b6, Wf, bf):
    src = edge_index[0].astype(jnp.int32)
    dst = edge_index[1].astype(jnp.int32)
    bat = batch.astype(jnp.int32)

    xp = jnp.pad(x, ((0, NP - N), (0, 0)))
    W1p = jnp.pad(W1, ((0, 0), (0, HN - H)))
    z1 = _mm_tc(xp, W1p)

    srcq = jnp.pad(src, (0, EPAD2 - E), constant_values=N).reshape(32, ECH2, 128)
    dstq = jnp.pad(dst, (0, EPAD2 - E), constant_values=N).reshape(32, ECH2, 128)
    batp = jnp.pad(bat, (0, NP - N), constant_values=G).reshape(NS, NCHN, 128)

    def dup8(W):
        Wp = jnp.pad(W, ((0, 0), (0, HN - W.shape[1])))
        return jnp.concatenate([Wp, Wp], axis=1)

    Wst = jnp.concatenate(
        [dup8(W) for W in (W2, W3, W4, W5, W6)]
        + [jnp.pad(Wf, ((0, 0), (0, HW - C)))], axis=0)

    def bdup(b):
        bp = jnp.pad(b, (0, HN - b.shape[0]))
        return jnp.concatenate([bp, bp])

    bst = jnp.stack([bdup(b) for b in (b1, b2, b3, b4, b5, b6)]
                    + [jnp.pad(bf, (0, HW - C))])

    sums, _, _, _ = _sc_kernel(z1, srcq, dstq, batp, Wst, bst)
    return _tail(sums, bat)
